# Initial kernel scaffold; baseline (speedup 1.0000x reference)
import jax
import jax.numpy as jnp
from jax.experimental import pallas as pl


def kernel(inputs, word_table, pos_table, depl_table, W1, b1, W2, b2, W3, b3):
    raise NotImplementedError("write your pallas kernel here")


# speedup vs baseline: 3.2454x; 3.2454x over previous
import functools

import jax
import jax.numpy as jnp
from jax import lax
from jax.experimental import pallas as pl
from jax.experimental.pallas import tpu as pltpu
from jax.experimental.pallas import tpu_sc as plsc

B = 16384
WORD_D, POS_D, DEPL_D = 64, 32, 32
N_WORD, N_POS, N_DEPL = 20, 20, 12
H1, H2, OUT = 512, 256, 128

NC, NS = 2, 16
NW = NC * NS
IDX_ROW = 128
GROUP = 8

WORD_ROWS = B * N_WORD // IDX_ROW
DEPL_ROWS = B * N_DEPL // IDX_ROW
WROWS_PW = WORD_ROWS // NW
DROWS_PW = DEPL_ROWS // NW
W_ITERS = WROWS_PW // GROUP
D_ITERS = DROWS_PW // GROUP


def _gather_body(word_idx, pos_idx, depl_idx, word_tab, pos_tab, depl_tab,
                 word_out, pos_out, depl_out, idx_v, wrows_v, prows_v, sem):
    wid = lax.axis_index("s") * NC + lax.axis_index("c")
    wbase = wid * WROWS_PW
    dbase = wid * DROWS_PW

    def table_loop(n_iters, base, idx_hbm, tab, out_hbm, rows_v):
        def step(t, carry):
            row0 = base + t * GROUP
            pltpu.sync_copy(idx_hbm.at[pl.ds(row0, GROUP)], idx_v)
            cps = [pltpu.async_copy(tab.at[idx_v.at[j]], rows_v.at[j], sem)
                   for j in range(GROUP)]
            for cp in cps:
                cp.wait()
            pltpu.sync_copy(rows_v, out_hbm.at[pl.ds(row0, GROUP)])
            return carry
        lax.fori_loop(0, n_iters, step, 0)

    table_loop(W_ITERS, wbase, word_idx, word_tab, word_out, wrows_v)
    table_loop(W_ITERS, wbase, pos_idx, pos_tab, pos_out, prows_v)
    table_loop(D_ITERS, dbase, depl_idx, depl_tab, depl_out, prows_v)


_gather = pl.kernel(
    _gather_body,
    out_type=(
        jax.ShapeDtypeStruct((WORD_ROWS, IDX_ROW, WORD_D), jnp.float32),
        jax.ShapeDtypeStruct((WORD_ROWS, IDX_ROW, POS_D), jnp.float32),
        jax.ShapeDtypeStruct((DEPL_ROWS, IDX_ROW, DEPL_D), jnp.float32),
    ),
    mesh=plsc.VectorSubcoreMesh(core_axis_name="c", subcore_axis_name="s",
                                num_cores=NC, num_subcores=NS),
    scratch_types=[
        pltpu.VMEM((GROUP, IDX_ROW), jnp.int32),
        pltpu.VMEM((GROUP, IDX_ROW, WORD_D), jnp.float32),
        pltpu.VMEM((GROUP, IDX_ROW, POS_D), jnp.float32),
        pltpu.SemaphoreType.DMA,
    ],
    compiler_params=pltpu.CompilerParams(use_tc_tiling_on_sc=False),
)


BM = 1024


def _mlp_body(we, pe, de, w1w, w1p, w1d, b1, w2, b2, w3, b3, out):
    h = jnp.dot(we[...], w1w[...], preferred_element_type=jnp.float32)
    h += jnp.dot(pe[...], w1p[...], preferred_element_type=jnp.float32)
    h += jnp.dot(de[...], w1d[...], preferred_element_type=jnp.float32)
    h += b1[...]
    h = jnp.where(h >= 0, h, 0.2 * h)
    h = jnp.dot(h, w2[...], preferred_element_type=jnp.float32) + b2[...]
    h = jnp.where(h >= 0, h, 0.2 * h)
    out[...] = jnp.dot(h, w3[...], preferred_element_type=jnp.float32) + b3[...]


def _mlp(we, pe, de, w1w, w1p, w1d, b1, w2, b2, w3, b3):
    full = lambda r, c: pl.BlockSpec((r, c), lambda i: (0, 0))
    return pl.pallas_call(
        _mlp_body,
        grid=(B // BM,),
        in_specs=[
            pl.BlockSpec((BM, N_WORD * WORD_D), lambda i: (i, 0)),
            pl.BlockSpec((BM, N_POS * POS_D), lambda i: (i, 0)),
            pl.BlockSpec((BM, N_DEPL * DEPL_D), lambda i: (i, 0)),
            full(N_WORD * WORD_D, H1),
            full(N_POS * POS_D, H1),
            full(N_DEPL * DEPL_D, H1),
            full(1, H1),
            full(H1, H2),
            full(1, H2),
            full(H2, OUT),
            full(1, OUT),
        ],
        out_specs=pl.BlockSpec((BM, OUT), lambda i: (i, 0)),
        out_shape=jax.ShapeDtypeStruct((B, OUT), jnp.float32),
    )(we, pe, de, w1w, w1p, w1d, b1, w2, b2, w3, b3)


def kernel(inputs, word_table, pos_table, depl_table, W1, b1, W2, b2, W3, b3):
    word_idx = inputs[:, 0:N_WORD].reshape(-1, IDX_ROW)
    pos_idx = inputs[:, N_WORD:N_WORD + N_POS].reshape(-1, IDX_ROW)
    depl_idx = inputs[:, N_WORD + N_POS:].reshape(-1, IDX_ROW)

    word_e, pos_e, depl_e = _gather(word_idx, pos_idx, depl_idx,
                                    word_table, pos_table, depl_table)

    we = word_e.reshape(B, N_WORD * WORD_D)
    pe = pos_e.reshape(B, N_POS * POS_D)
    de = depl_e.reshape(B, N_DEPL * DEPL_D)

    c1 = N_WORD * WORD_D
    c2 = c1 + N_POS * POS_D
    return _mlp(we, pe, de, W1[:c1], W1[c1:c2], W1[c2:],
                b1.reshape(1, H1), W2, b2.reshape(1, H2),
                W3, b3.reshape(1, OUT))
